# baseline (speedup 1.0000x reference)
import jax
import jax.numpy as jnp
from jax.experimental import pallas as pl
from jax.experimental.pallas import tpu as pltpu

_BM = 256
_NS = 4


def _gate_gemm_kernel(xa, xb, xc, xd, w_ref, ot_ref):
    for idx, xr in enumerate((xa, xb, xc, xd)):
        ot_ref[:, idx * _BM:(idx + 1) * _BM] = jax.lax.dot_general(
            w_ref[...], xr[...],
            dimension_numbers=(((1,), (1,)), ((), ())),
            preferred_element_type=jnp.float32)


def kernel(hidden_states, weight):
    m, k = hidden_states.shape
    e = weight.shape[0]

    def mk(off):
        return pl.BlockSpec((_BM, k), lambda i, off=off: (_NS * i + off, 0))

    out_t = pl.pallas_call(
        _gate_gemm_kernel,
        grid=(m // (_NS * _BM),),
        in_specs=[mk(0), mk(1), mk(2), mk(3),
                  pl.BlockSpec((e, k), lambda i: (0, 0))],
        out_specs=pl.BlockSpec((e, _NS * _BM), lambda i: (0, i)),
        out_shape=jax.ShapeDtypeStruct((e, m), jnp.float32),
        compiler_params=pltpu.CompilerParams(
            dimension_semantics=("arbitrary",),
        ),
    )(hidden_states, hidden_states, hidden_states, hidden_states, weight)
    return out_t.T


# speedup vs baseline: 1.0019x; 1.0019x over previous
import jax
import jax.numpy as jnp
from jax.experimental import pallas as pl
from jax.experimental.pallas import tpu as pltpu

_BM = 1024


def _gate_gemm_kernel(x_ref, w_ref, ot_ref):
    ot_ref[...] = jax.lax.dot_general(
        w_ref[...], x_ref[...],
        dimension_numbers=(((1,), (1,)), ((), ())),
        preferred_element_type=jnp.float32)


def kernel(hidden_states, weight):
    m, k = hidden_states.shape
    e = weight.shape[0]
    out_t = pl.pallas_call(
        _gate_gemm_kernel,
        grid=(pl.cdiv(m, _BM),),
        in_specs=[
            pl.BlockSpec((_BM, k), lambda i: (i, 0)),
            pl.BlockSpec((e, k), lambda i: (0, 0)),
        ],
        out_specs=pl.BlockSpec((e, _BM), lambda i: (0, i)),
        out_shape=jax.ShapeDtypeStruct((e, m), jnp.float32),
        compiler_params=pltpu.CompilerParams(
            dimension_semantics=("arbitrary",),
        ),
    )(hidden_states, weight)
    return out_t.T
